# Initial kernel scaffold; baseline (speedup 1.0000x reference)
import functools

import jax
import jax.numpy as jnp
from jax import lax
from jax.experimental import pallas as pl
from jax.experimental.pallas import tpu as pltpu
from jax.experimental.pallas import tpu_sc as plsc

N = 10000
E = 320000
NP = 10240
EPT = 10240
EP = EPT * 32
ROWS_PT = EPT // 128
SUPR = 8
NSUP = ROWS_PT // SUPR
RP_SC = NP // 16

_mesh = plsc.VectorSubcoreMesh(core_axis_name="c", subcore_axis_name="s")


def _deg_kernel(col2d):
    @functools.partial(
        pl.kernel,
        out_type=jax.ShapeDtypeStruct((2, NP), jnp.float32),
        mesh=_mesh,
        scratch_types=[
            pltpu.VMEM((SUPR, 128), jnp.int32),
            pltpu.VMEM((128,), jnp.float32),
            pltpu.VMEM((RP_SC,), jnp.float32),
            pltpu.VMEM_SHARED((NP,), jnp.float32),
            pltpu.SemaphoreType.DMA,
        ],
    )
    def k(col_hbm, out_hbm, cix, ones, zbuf, acc, sem):
        cid = lax.axis_index("c")
        sid = lax.axis_index("s")
        tid = cid * 16 + sid

        @pl.loop(0, 128 // 16)
        def _(i):
            ones[pl.ds(i * 16, 16)] = jnp.ones((16,), jnp.float32)

        @pl.loop(0, RP_SC // 16)
        def _(i):
            zbuf[pl.ds(i * 16, 16)] = jnp.zeros((16,), jnp.float32)

        pltpu.sync_copy(zbuf, acc.at[pl.ds(sid * RP_SC, RP_SC)])
        plsc.subcore_barrier()

        ibase = tid * ROWS_PT

        @pl.loop(0, NSUP)
        def _(s):
            pltpu.sync_copy(col_hbm.at[pl.ds(ibase + s * SUPR, SUPR)], cix)
            for j in range(SUPR):
                pltpu.sync_copy(ones, acc.at[cix.at[j]], add=True)

        plsc.subcore_barrier()
        pltpu.sync_copy(
            acc.at[pl.ds(sid * RP_SC, RP_SC)],
            out_hbm.at[cid].at[pl.ds(sid * RP_SC, RP_SC)],
        )

    return k(col2d)


def _edge_layer(hs, row2d, col2d, D):
    @functools.partial(
        pl.kernel,
        out_type=jax.ShapeDtypeStruct((2, NP, D), jnp.float32),
        mesh=_mesh,
        scratch_types=[
            pltpu.VMEM((SUPR, 128), jnp.int32),
            pltpu.VMEM((SUPR, 128), jnp.int32),
            pltpu.VMEM((SUPR * 128, D), jnp.float32),
            pltpu.VMEM((16, D), jnp.float32),
            pltpu.VMEM_SHARED((NP, D), jnp.float32),
            pltpu.SemaphoreType.DMA,
        ],
    )
    def k(hs_hbm, row_hbm, col_hbm, out_hbm, rix, cix, msg, zblk, acc, gsem):
        cid = lax.axis_index("c")
        sid = lax.axis_index("s")
        tid = cid * 16 + sid

        for i in range(16):
            for j in range(D // 16):
                zblk[i, pl.ds(j * 16, 16)] = jnp.zeros((16,), jnp.float32)

        rbase = sid * RP_SC

        @pl.loop(0, RP_SC // 16)
        def _(i):
            pltpu.sync_copy(zblk, acc.at[pl.ds(rbase + i * 16, 16)])

        plsc.subcore_barrier()

        ibase = tid * ROWS_PT

        @pl.loop(0, NSUP)
        def _(s):
            ro = ibase + s * SUPR
            pltpu.sync_copy(row_hbm.at[pl.ds(ro, SUPR)], rix)
            pltpu.sync_copy(col_hbm.at[pl.ds(ro, SUPR)], cix)
            copies = [
                pltpu.async_copy(
                    hs_hbm.at[rix.at[j]], msg.at[pl.ds(j * 128, 128)], gsem
                )
                for j in range(SUPR)
            ]
            for c in copies:
                c.wait()
            for j in range(SUPR):
                pltpu.sync_copy(
                    msg.at[pl.ds(j * 128, 128)], acc.at[cix.at[j]], add=True
                )

        plsc.subcore_barrier()
        pltpu.sync_copy(
            acc.at[pl.ds(rbase, RP_SC)],
            out_hbm.at[cid].at[pl.ds(rbase, RP_SC)],
        )

    return k(hs, row2d, col2d)


def _dis_of(degp):
    d = degp[0, :N, :] + degp[1, :N, :] + 1.0
    return lax.rsqrt(d)


def _mmul(a, b):
    return jnp.dot(a, b, preferred_element_type=jnp.float32,
                   precision=lax.Precision.HIGHEST)


def _bnorm(h, g, b):
    m = jnp.mean(h, axis=0, keepdims=True)
    hc = h - m
    v = jnp.mean(hc * hc, axis=0, keepdims=True)
    return hc * lax.rsqrt(v + 1e-5) * g + b


def _tc_mm0(x, w):
    def body(x_ref, w_ref, o_ref):
        o_ref[...] = _mmul(x_ref[...], w_ref[...])

    return pl.pallas_call(
        body,
        out_shape=jax.ShapeDtypeStruct((x.shape[0], w.shape[1]), jnp.float32),
    )(x, w)


def _tc_scale0(hw0, degp):
    def body(hw_ref, dg_ref, o_ref):
        dis = _dis_of(dg_ref[...])
        o_ref[...] = hw_ref[...] * dis

    return pl.pallas_call(
        body,
        out_shape=jax.ShapeDtypeStruct(hw0.shape, jnp.float32),
    )(hw0, degp)


def _tc_layer(part, hw, degp, b, g, beta, w_next):
    def body(p_ref, hw_ref, dg_ref, b_ref, g_ref, be_ref, w_ref, hs_o, hw_o):
        dis = _dis_of(dg_ref[...])
        scat = p_ref[0, :N, :] + p_ref[1, :N, :]
        pre = dis * scat + (dis * dis) * hw_ref[...] + b_ref[...]
        h = _bnorm(jnp.maximum(pre, 0.0), g_ref[...], be_ref[...])
        hw_n = _mmul(h, w_ref[...])
        hw_o[...] = hw_n
        hs_o[...] = dis * hw_n

    F = w_next.shape[1]
    return pl.pallas_call(
        body,
        out_shape=(
            jax.ShapeDtypeStruct((N, F), jnp.float32),
            jax.ShapeDtypeStruct((N, F), jnp.float32),
        ),
    )(part, hw, degp, b.reshape(1, -1), g.reshape(1, -1), beta.reshape(1, -1),
      w_next)


def _tc_tail(part, hw, degp, b, g, beta, ws):
    (lat_W, lat_b, dec_W0, dec_b0, dg0, db0, dec_W1, dec_b1, dg1, db1,
     dec_W2, dec_b2, dg2, db2, out_W, out_b) = ws

    def body(p_ref, hw_ref, dg_ref, b_ref, g_ref, be_ref,
             lw_ref, lb_ref, w0_ref, c0_ref, g0_ref, be0_ref,
             w1_ref, c1_ref, g1_ref, be1_ref,
             w2_ref, c2_ref, g2_ref, be2_ref, ow_ref, ob_ref, o_ref):
        dis = _dis_of(dg_ref[...])
        scat = p_ref[0, :N, :] + p_ref[1, :N, :]
        pre = dis * scat + (dis * dis) * hw_ref[...] + b_ref[...]
        h = _bnorm(jnp.maximum(pre, 0.0), g_ref[...], be_ref[...])
        z = _mmul(h, lw_ref[...]) + lb_ref[...]
        d = _bnorm(jnp.maximum(_mmul(z, w0_ref[...]) + c0_ref[...], 0.0),
                   g0_ref[...], be0_ref[...])
        d = _bnorm(jnp.maximum(_mmul(d, w1_ref[...]) + c1_ref[...], 0.0),
                   g1_ref[...], be1_ref[...])
        d = _bnorm(jnp.maximum(_mmul(d, w2_ref[...]) + c2_ref[...], 0.0),
                   g2_ref[...], be2_ref[...])
        o_ref[...] = _mmul(d, ow_ref[...]) + ob_ref[...]

    r = lambda a: a.reshape(1, -1)
    return pl.pallas_call(
        body,
        out_shape=jax.ShapeDtypeStruct((N, out_W.shape[1]), jnp.float32),
    )(part, hw, degp, r(b), r(g), r(beta),
      lat_W, r(lat_b), dec_W0, r(dec_b0), r(dg0), r(db0),
      dec_W1, r(dec_b1), r(dg1), r(db1),
      dec_W2, r(dec_b2), r(dg2), r(db2), out_W, r(out_b))


def kernel(x, edge_index, enc_W0, enc_b0, bn_g0, bn_b0, enc_W1, enc_b1,
           bn_g1, bn_b1, enc_W2, enc_b2, bn_g2, bn_b2, lat_W, lat_b,
           dec_W0, dec_b0, dbn_g0, dbn_b0, dec_W1, dec_b1, dbn_g1, dbn_b1,
           dec_W2, dec_b2, dbn_g2, dbn_b2, out_W, out_b):
    npad = EP - E
    pi = jnp.arange(npad, dtype=jnp.int32)
    row = jnp.concatenate([edge_index[0], pi % N])
    col = jnp.concatenate([edge_index[1], N + pi % (NP - N)])
    row2d = row.reshape(EP // 128, 128)
    col2d = col.reshape(EP // 128, 128)

    degp = _deg_kernel(col2d)
    hw0 = _tc_mm0(x, enc_W0)
    degp = degp.reshape(2, NP, 1)

    hs0 = _tc_scale0(hw0, degp)
    p0 = _edge_layer(hs0, row2d, col2d, 64)
    hs1, hw1 = _tc_layer(p0, hw0, degp, enc_b0, bn_g0, bn_b0, enc_W1)
    p1 = _edge_layer(hs1, row2d, col2d, 32)
    hs2, hw2 = _tc_layer(p1, hw1, degp, enc_b1, bn_g1, bn_b1, enc_W2)
    p2 = _edge_layer(hs2, row2d, col2d, 16)
    return _tc_tail(p2, hw2, degp, enc_b2, bn_g2, bn_b2,
                    (lat_W, lat_b, dec_W0, dec_b0, dbn_g0, dbn_b0,
                     dec_W1, dec_b1, dbn_g1, dbn_b1,
                     dec_W2, dec_b2, dbn_g2, dbn_b2, out_W, out_b))


# speedup vs baseline: 31.9562x; 31.9562x over previous
import functools

import jax
import jax.numpy as jnp
from jax import lax
from jax.experimental import pallas as pl
from jax.experimental.pallas import tpu as pltpu
from jax.experimental.pallas import tpu_sc as plsc

N = 10000
E = 320000
NP = 10240
EPT = 10240
EP = EPT * 32
ROWS_PT = EPT // 128
SUPR = 8
NSUP = ROWS_PT // SUPR
RP_SC = NP // 16

_mesh = plsc.VectorSubcoreMesh(core_axis_name="c", subcore_axis_name="s")
_sc_params = pltpu.CompilerParams(use_tc_tiling_on_sc=False)
_tc_params = pltpu.CompilerParams(vmem_limit_bytes=64 * 1024 * 1024)


def _deg_kernel(col2d):
    @functools.partial(
        pl.kernel,
        out_type=jax.ShapeDtypeStruct((2, NP), jnp.float32),
        mesh=_mesh,
        compiler_params=_sc_params,
        scratch_types=[
            pltpu.VMEM((SUPR, 128), jnp.int32),
            pltpu.VMEM((128,), jnp.float32),
            pltpu.VMEM((RP_SC,), jnp.float32),
            pltpu.VMEM_SHARED((NP,), jnp.float32),
            pltpu.SemaphoreType.DMA,
        ],
    )
    def k(col_hbm, out_hbm, cix, ones, zbuf, acc, sem):
        cid = lax.axis_index("c")
        sid = lax.axis_index("s")
        tid = cid * 16 + sid

        @pl.loop(0, 128 // 16)
        def _(i):
            ones[pl.ds(i * 16, 16)] = jnp.ones((16,), jnp.float32)

        @pl.loop(0, RP_SC // 16)
        def _(i):
            zbuf[pl.ds(i * 16, 16)] = jnp.zeros((16,), jnp.float32)

        pltpu.sync_copy(zbuf, acc.at[pl.ds(sid * RP_SC, RP_SC)])
        plsc.subcore_barrier()

        ibase = tid * ROWS_PT

        @pl.loop(0, NSUP)
        def _(s):
            pltpu.sync_copy(col_hbm.at[pl.ds(ibase + s * SUPR, SUPR)], cix)
            for j in range(SUPR):
                pltpu.sync_copy(ones, acc.at[cix.at[j]], add=True)

        plsc.subcore_barrier()
        pltpu.sync_copy(
            acc.at[pl.ds(sid * RP_SC, RP_SC)],
            out_hbm.at[cid].at[pl.ds(sid * RP_SC, RP_SC)],
        )

    return k(col2d)


def _edge_layer(hs, row2d, col2d, D):
    @functools.partial(
        pl.kernel,
        out_type=jax.ShapeDtypeStruct((2, NP, D), jnp.float32),
        mesh=_mesh,
        compiler_params=_sc_params,
        scratch_types=[
            pltpu.VMEM((SUPR, 128), jnp.int32),
            pltpu.VMEM((SUPR, 128), jnp.int32),
            pltpu.VMEM((SUPR * 128, D), jnp.float32),
            pltpu.VMEM((16, D), jnp.float32),
            pltpu.VMEM_SHARED((NP, D), jnp.float32),
            pltpu.SemaphoreType.DMA,
        ],
    )
    def k(hs_hbm, row_hbm, col_hbm, out_hbm, rix, cix, msg, zblk, acc, gsem):
        cid = lax.axis_index("c")
        sid = lax.axis_index("s")
        tid = cid * 16 + sid

        for i in range(16):
            for j in range(D // 16):
                zblk[i, pl.ds(j * 16, 16)] = jnp.zeros((16,), jnp.float32)

        rbase = sid * RP_SC

        @pl.loop(0, RP_SC // 16)
        def _(i):
            pltpu.sync_copy(zblk, acc.at[pl.ds(rbase + i * 16, 16)])

        plsc.subcore_barrier()

        ibase = tid * ROWS_PT

        @pl.loop(0, NSUP)
        def _(s):
            ro = ibase + s * SUPR
            pltpu.sync_copy(row_hbm.at[pl.ds(ro, SUPR)], rix)
            pltpu.sync_copy(col_hbm.at[pl.ds(ro, SUPR)], cix)
            copies = [
                pltpu.async_copy(
                    hs_hbm.at[rix.at[j]], msg.at[pl.ds(j * 128, 128)], gsem
                )
                for j in range(SUPR)
            ]
            for c in copies:
                c.wait()
            for j in range(SUPR):
                pltpu.sync_copy(
                    msg.at[pl.ds(j * 128, 128)], acc.at[cix.at[j]], add=True
                )

        plsc.subcore_barrier()
        pltpu.sync_copy(
            acc.at[pl.ds(rbase, RP_SC)],
            out_hbm.at[cid].at[pl.ds(rbase, RP_SC)],
        )

    return k(hs, row2d, col2d)


def _dis_of(degp):
    d = degp[0, :N, :] + degp[1, :N, :] + 1.0
    return lax.rsqrt(d)


def _mmul(a, b):
    return jnp.dot(a, b, preferred_element_type=jnp.float32)


def _bnorm(h, g, b):
    m = jnp.mean(h, axis=0, keepdims=True)
    hc = h - m
    v = jnp.mean(hc * hc, axis=0, keepdims=True)
    return hc * lax.rsqrt(v + 1e-5) * g + b


def _tc_mm0(x, w):
    def body(x_ref, w_ref, o_ref):
        o_ref[...] = _mmul(x_ref[...], w_ref[...])

    return pl.pallas_call(
        body,
        compiler_params=_tc_params,
        out_shape=jax.ShapeDtypeStruct((x.shape[0], w.shape[1]), jnp.float32),
    )(x, w)


def _tc_scale0(hw0, degp):
    def body(hw_ref, dg_ref, o_ref):
        dis = _dis_of(dg_ref[...])
        o_ref[...] = hw_ref[...] * dis

    return pl.pallas_call(
        body,
        compiler_params=_tc_params,
        out_shape=jax.ShapeDtypeStruct(hw0.shape, jnp.float32),
    )(hw0, degp)


def _tc_layer(part, hw, degp, b, g, beta, w_next):
    def body(p_ref, hw_ref, dg_ref, b_ref, g_ref, be_ref, w_ref, hs_o, hw_o):
        dis = _dis_of(dg_ref[...])
        scat = p_ref[0, :N, :] + p_ref[1, :N, :]
        pre = dis * scat + (dis * dis) * hw_ref[...] + b_ref[...]
        h = _bnorm(jnp.maximum(pre, 0.0), g_ref[...], be_ref[...])
        hw_n = _mmul(h, w_ref[...])
        hw_o[...] = hw_n
        hs_o[...] = dis * hw_n

    F = w_next.shape[1]
    return pl.pallas_call(
        body,
        compiler_params=_tc_params,
        out_shape=(
            jax.ShapeDtypeStruct((N, F), jnp.float32),
            jax.ShapeDtypeStruct((N, F), jnp.float32),
        ),
    )(part, hw, degp, b.reshape(1, -1), g.reshape(1, -1), beta.reshape(1, -1),
      w_next)


def _tc_tail(part, hw, degp, b, g, beta, ws):
    (lat_W, lat_b, dec_W0, dec_b0, dg0, db0, dec_W1, dec_b1, dg1, db1,
     dec_W2, dec_b2, dg2, db2, out_W, out_b) = ws

    def body(p_ref, hw_ref, dg_ref, b_ref, g_ref, be_ref,
             lw_ref, lb_ref, w0_ref, c0_ref, g0_ref, be0_ref,
             w1_ref, c1_ref, g1_ref, be1_ref,
             w2_ref, c2_ref, g2_ref, be2_ref, ow_ref, ob_ref, o_ref):
        dis = _dis_of(dg_ref[...])
        scat = p_ref[0, :N, :] + p_ref[1, :N, :]
        pre = dis * scat + (dis * dis) * hw_ref[...] + b_ref[...]
        h = _bnorm(jnp.maximum(pre, 0.0), g_ref[...], be_ref[...])
        z = _mmul(h, lw_ref[...]) + lb_ref[...]
        d = _bnorm(jnp.maximum(_mmul(z, w0_ref[...]) + c0_ref[...], 0.0),
                   g0_ref[...], be0_ref[...])
        d = _bnorm(jnp.maximum(_mmul(d, w1_ref[...]) + c1_ref[...], 0.0),
                   g1_ref[...], be1_ref[...])
        d = _bnorm(jnp.maximum(_mmul(d, w2_ref[...]) + c2_ref[...], 0.0),
                   g2_ref[...], be2_ref[...])
        o_ref[...] = _mmul(d, ow_ref[...]) + ob_ref[...]

    r = lambda a: a.reshape(1, -1)
    return pl.pallas_call(
        body,
        compiler_params=_tc_params,
        out_shape=jax.ShapeDtypeStruct((N, out_W.shape[1]), jnp.float32),
    )(part, hw, degp, r(b), r(g), r(beta),
      lat_W, r(lat_b), dec_W0, r(dec_b0), r(dg0), r(db0),
      dec_W1, r(dec_b1), r(dg1), r(db1),
      dec_W2, r(dec_b2), r(dg2), r(db2), out_W, r(out_b))


def kernel(x, edge_index, enc_W0, enc_b0, bn_g0, bn_b0, enc_W1, enc_b1,
           bn_g1, bn_b1, enc_W2, enc_b2, bn_g2, bn_b2, lat_W, lat_b,
           dec_W0, dec_b0, dbn_g0, dbn_b0, dec_W1, dec_b1, dbn_g1, dbn_b1,
           dec_W2, dec_b2, dbn_g2, dbn_b2, out_W, out_b):
    npad = EP - E
    pi = jnp.arange(npad, dtype=jnp.int32)
    row = jnp.concatenate([edge_index[0], pi % N])
    col = jnp.concatenate([edge_index[1], N + pi % (NP - N)])
    row2d = row.reshape(EP // 128, 128)
    col2d = col.reshape(EP // 128, 128)

    degp = _deg_kernel(col2d)
    hw0 = _tc_mm0(x, enc_W0)
    degp = degp.reshape(2, NP, 1)

    hs0 = _tc_scale0(hw0, degp)
    p0 = _edge_layer(hs0, row2d, col2d, 64)
    hs1, hw1 = _tc_layer(p0, hw0, degp, enc_b0, bn_g0, bn_b0, enc_W1)
    p1 = _edge_layer(hs1, row2d, col2d, 32)
    hs2, hw2 = _tc_layer(p1, hw1, degp, enc_b1, bn_g1, bn_b1, enc_W2)
    p2 = _edge_layer(hs2, row2d, col2d, 16)
    return _tc_tail(p2, hw2, degp, enc_b2, bn_g2, bn_b2,
                    (lat_W, lat_b, dec_W0, dec_b0, dbn_g0, dbn_b0,
                     dec_W1, dec_b1, dbn_g1, dbn_b1,
                     dec_W2, dec_b2, dbn_g2, dbn_b2, out_W, out_b))


# speedup vs baseline: 35.8748x; 1.1226x over previous
import functools

import jax
import jax.numpy as jnp
from jax import lax
from jax.experimental import pallas as pl
from jax.experimental.pallas import tpu as pltpu
from jax.experimental.pallas import tpu_sc as plsc

N = 10000
E = 320000
NP = 10240
EPT = 10240
EP = EPT * 32
ROWS_PT = EPT // 128
SUPR = 8
NSUP = ROWS_PT // SUPR
CHR = 4
NCH = ROWS_PT // CHR
RP_SC = NP // 16

_mesh = plsc.VectorSubcoreMesh(core_axis_name="c", subcore_axis_name="s")
_sc_params = pltpu.CompilerParams(use_tc_tiling_on_sc=False)
_tc_params = pltpu.CompilerParams(vmem_limit_bytes=64 * 1024 * 1024)


def _deg_kernel(col2d):
    @functools.partial(
        pl.kernel,
        out_type=jax.ShapeDtypeStruct((2, NP), jnp.float32),
        mesh=_mesh,
        compiler_params=_sc_params,
        scratch_types=[
            pltpu.VMEM((SUPR, 128), jnp.int32),
            pltpu.VMEM((128,), jnp.float32),
            pltpu.VMEM((RP_SC,), jnp.float32),
            pltpu.VMEM_SHARED((NP,), jnp.float32),
            pltpu.SemaphoreType.DMA,
        ],
    )
    def k(col_hbm, out_hbm, cix, ones, zbuf, acc, sem):
        cid = lax.axis_index("c")
        sid = lax.axis_index("s")
        tid = cid * 16 + sid

        @pl.loop(0, 128 // 16)
        def _(i):
            ones[pl.ds(i * 16, 16)] = jnp.ones((16,), jnp.float32)

        @pl.loop(0, RP_SC // 16)
        def _(i):
            zbuf[pl.ds(i * 16, 16)] = jnp.zeros((16,), jnp.float32)

        pltpu.sync_copy(zbuf, acc.at[pl.ds(sid * RP_SC, RP_SC)])
        plsc.subcore_barrier()

        ibase = tid * ROWS_PT

        @pl.loop(0, NSUP)
        def _(s):
            pltpu.sync_copy(col_hbm.at[pl.ds(ibase + s * SUPR, SUPR)], cix)
            for j in range(SUPR):
                pltpu.sync_copy(ones, acc.at[cix.at[j]], add=True)

        plsc.subcore_barrier()
        pltpu.sync_copy(
            acc.at[pl.ds(sid * RP_SC, RP_SC)],
            out_hbm.at[cid].at[pl.ds(sid * RP_SC, RP_SC)],
        )

    return k(col2d)


def _edge_layer(hs, row2d, col2d, D):
    @functools.partial(
        pl.kernel,
        out_type=jax.ShapeDtypeStruct((2, NP, D), jnp.float32),
        mesh=_mesh,
        compiler_params=_sc_params,
        scratch_types=[
            pltpu.VMEM((2, CHR, 128), jnp.int32),
            pltpu.VMEM((2, CHR, 128), jnp.int32),
            pltpu.VMEM((2, CHR * 128, D), jnp.float32),
            pltpu.VMEM((16, D), jnp.float32),
            pltpu.VMEM_SHARED((NP, D), jnp.float32),
            pltpu.SemaphoreType.DMA,
        ],
    )
    def k(hs_hbm, row_hbm, col_hbm, out_hbm, rix, cix, msg, zblk, acc, gsem):
        cid = lax.axis_index("c")
        sid = lax.axis_index("s")
        tid = cid * 16 + sid

        for i in range(16):
            for j in range(D // 16):
                zblk[i, pl.ds(j * 16, 16)] = jnp.zeros((16,), jnp.float32)

        rbase = sid * RP_SC

        @pl.loop(0, RP_SC // 16)
        def _(i):
            pltpu.sync_copy(zblk, acc.at[pl.ds(rbase + i * 16, 16)])

        plsc.subcore_barrier()

        ibase = tid * ROWS_PT

        def fire(slot, s):
            ro = ibase + s * CHR
            pltpu.sync_copy(row_hbm.at[pl.ds(ro, CHR)], rix.at[slot])
            pltpu.sync_copy(col_hbm.at[pl.ds(ro, CHR)], cix.at[slot])
            for j in range(CHR):
                pltpu.async_copy(
                    hs_hbm.at[rix.at[slot].at[j]],
                    msg.at[slot].at[pl.ds(j * 128, 128)],
                    gsem,
                )

        fire(0, 0)
        fire(1, 1)

        @pl.loop(0, NCH, step=2)
        def _(s0):
            for b in range(2):
                s = s0 + b
                for j in range(CHR):
                    pltpu.make_async_copy(
                        hs_hbm.at[rix.at[b].at[j]],
                        msg.at[b].at[pl.ds(j * 128, 128)],
                        gsem,
                    ).wait()

                for j in range(CHR):
                    pltpu.sync_copy(
                        msg.at[b].at[pl.ds(j * 128, 128)],
                        acc.at[cix.at[b].at[j]],
                        add=True,
                    )

                @pl.when(s + 2 < NCH)
                def _():
                    fire(b, s + 2)

        plsc.subcore_barrier()
        pltpu.sync_copy(
            acc.at[pl.ds(rbase, RP_SC)],
            out_hbm.at[cid].at[pl.ds(rbase, RP_SC)],
        )

    return k(hs, row2d, col2d)


def _dis_of(degp):
    d = degp[0, :N, :] + degp[1, :N, :] + 1.0
    return lax.rsqrt(d)


def _mmul(a, b):
    return jnp.dot(a, b, preferred_element_type=jnp.float32)


def _bnorm(h, g, b):
    m = jnp.mean(h, axis=0, keepdims=True)
    hc = h - m
    v = jnp.mean(hc * hc, axis=0, keepdims=True)
    return hc * lax.rsqrt(v + 1e-5) * g + b


def _tc_mm0(x, w):
    def body(x_ref, w_ref, o_ref):
        o_ref[...] = _mmul(x_ref[...], w_ref[...])

    return pl.pallas_call(
        body,
        compiler_params=_tc_params,
        out_shape=jax.ShapeDtypeStruct((x.shape[0], w.shape[1]), jnp.float32),
    )(x, w)


def _tc_scale0(hw0, degp):
    def body(hw_ref, dg_ref, o_ref):
        dis = _dis_of(dg_ref[...])
        o_ref[...] = hw_ref[...] * dis

    return pl.pallas_call(
        body,
        compiler_params=_tc_params,
        out_shape=jax.ShapeDtypeStruct(hw0.shape, jnp.float32),
    )(hw0, degp)


def _tc_layer(part, hw, degp, b, g, beta, w_next):
    def body(p_ref, hw_ref, dg_ref, b_ref, g_ref, be_ref, w_ref, hs_o, hw_o):
        dis = _dis_of(dg_ref[...])
        scat = p_ref[0, :N, :] + p_ref[1, :N, :]
        pre = dis * scat + (dis * dis) * hw_ref[...] + b_ref[...]
        h = _bnorm(jnp.maximum(pre, 0.0), g_ref[...], be_ref[...])
        hw_n = _mmul(h, w_ref[...])
        hw_o[...] = hw_n
        hs_o[...] = dis * hw_n

    F = w_next.shape[1]
    return pl.pallas_call(
        body,
        compiler_params=_tc_params,
        out_shape=(
            jax.ShapeDtypeStruct((N, F), jnp.float32),
            jax.ShapeDtypeStruct((N, F), jnp.float32),
        ),
    )(part, hw, degp, b.reshape(1, -1), g.reshape(1, -1), beta.reshape(1, -1),
      w_next)


def _tc_tail(part, hw, degp, b, g, beta, ws):
    (lat_W, lat_b, dec_W0, dec_b0, dg0, db0, dec_W1, dec_b1, dg1, db1,
     dec_W2, dec_b2, dg2, db2, out_W, out_b) = ws

    def body(p_ref, hw_ref, dg_ref, b_ref, g_ref, be_ref,
             lw_ref, lb_ref, w0_ref, c0_ref, g0_ref, be0_ref,
             w1_ref, c1_ref, g1_ref, be1_ref,
             w2_ref, c2_ref, g2_ref, be2_ref, ow_ref, ob_ref, o_ref):
        dis = _dis_of(dg_ref[...])
        scat = p_ref[0, :N, :] + p_ref[1, :N, :]
        pre = dis * scat + (dis * dis) * hw_ref[...] + b_ref[...]
        h = _bnorm(jnp.maximum(pre, 0.0), g_ref[...], be_ref[...])
        z = _mmul(h, lw_ref[...]) + lb_ref[...]
        d = _bnorm(jnp.maximum(_mmul(z, w0_ref[...]) + c0_ref[...], 0.0),
                   g0_ref[...], be0_ref[...])
        d = _bnorm(jnp.maximum(_mmul(d, w1_ref[...]) + c1_ref[...], 0.0),
                   g1_ref[...], be1_ref[...])
        d = _bnorm(jnp.maximum(_mmul(d, w2_ref[...]) + c2_ref[...], 0.0),
                   g2_ref[...], be2_ref[...])
        o_ref[...] = _mmul(d, ow_ref[...]) + ob_ref[...]

    r = lambda a: a.reshape(1, -1)
    return pl.pallas_call(
        body,
        compiler_params=_tc_params,
        out_shape=jax.ShapeDtypeStruct((N, out_W.shape[1]), jnp.float32),
    )(part, hw, degp, r(b), r(g), r(beta),
      lat_W, r(lat_b), dec_W0, r(dec_b0), r(dg0), r(db0),
      dec_W1, r(dec_b1), r(dg1), r(db1),
      dec_W2, r(dec_b2), r(dg2), r(db2), out_W, r(out_b))


def kernel(x, edge_index, enc_W0, enc_b0, bn_g0, bn_b0, enc_W1, enc_b1,
           bn_g1, bn_b1, enc_W2, enc_b2, bn_g2, bn_b2, lat_W, lat_b,
           dec_W0, dec_b0, dbn_g0, dbn_b0, dec_W1, dec_b1, dbn_g1, dbn_b1,
           dec_W2, dec_b2, dbn_g2, dbn_b2, out_W, out_b):
    npad = EP - E
    pi = jnp.arange(npad, dtype=jnp.int32)
    row = jnp.concatenate([edge_index[0], pi % N])
    col = jnp.concatenate([edge_index[1], N + pi % (NP - N)])
    row2d = row.reshape(EP // 128, 128)
    col2d = col.reshape(EP // 128, 128)

    degp = _deg_kernel(col2d)
    hw0 = _tc_mm0(x, enc_W0)
    degp = degp.reshape(2, NP, 1)

    hs0 = _tc_scale0(hw0, degp)
    p0 = _edge_layer(hs0, row2d, col2d, 64)
    hs1, hw1 = _tc_layer(p0, hw0, degp, enc_b0, bn_g0, bn_b0, enc_W1)
    p1 = _edge_layer(hs1, row2d, col2d, 32)
    hs2, hw2 = _tc_layer(p1, hw1, degp, enc_b1, bn_g1, bn_b1, enc_W2)
    p2 = _edge_layer(hs2, row2d, col2d, 16)
    return _tc_tail(p2, hw2, degp, enc_b2, bn_g2, bn_b2,
                    (lat_W, lat_b, dec_W0, dec_b0, dbn_g0, dbn_b0,
                     dec_W1, dec_b1, dbn_g1, dbn_b1,
                     dec_W2, dec_b2, dbn_g2, dbn_b2, out_W, out_b))


# speedup vs baseline: 37.2080x; 1.0372x over previous
import functools

import jax
import jax.numpy as jnp
from jax import lax
from jax.experimental import pallas as pl
from jax.experimental.pallas import tpu as pltpu
from jax.experimental.pallas import tpu_sc as plsc

N = 10000
E = 320000
NP = 10240
EPT = 10240
EP = EPT * 32
ROWS_PT = EPT // 128
SUPR = 8
NSUP = ROWS_PT // SUPR
CHR = 4
NCH = ROWS_PT // CHR
RP_SC = NP // 16

_mesh = plsc.VectorSubcoreMesh(core_axis_name="c", subcore_axis_name="s")
_sc_params = pltpu.CompilerParams(use_tc_tiling_on_sc=False)
_tc_params = pltpu.CompilerParams(vmem_limit_bytes=64 * 1024 * 1024)


def _deg_kernel(col2d):
    @functools.partial(
        pl.kernel,
        out_type=jax.ShapeDtypeStruct((2, NP), jnp.float32),
        mesh=_mesh,
        compiler_params=_sc_params,
        scratch_types=[
            pltpu.VMEM((2, SUPR, 128), jnp.int32),
            pltpu.VMEM((128,), jnp.float32),
            pltpu.VMEM((RP_SC,), jnp.float32),
            pltpu.VMEM_SHARED((NP,), jnp.float32),
            pltpu.SemaphoreType.DMA,
        ],
    )
    def k(col_hbm, out_hbm, cix, ones, zbuf, acc, isem):
        cid = lax.axis_index("c")
        sid = lax.axis_index("s")
        tid = cid * 16 + sid
        ibase = tid * ROWS_PT

        def fire(slot, s):
            pltpu.async_copy(
                col_hbm.at[pl.ds(ibase + s * SUPR, SUPR)], cix.at[slot], isem
            )

        fire(0, 0)
        fire(1, 1)

        @pl.loop(0, 128 // 16)
        def _(i):
            ones[pl.ds(i * 16, 16)] = jnp.ones((16,), jnp.float32)

        @pl.loop(0, RP_SC // 16)
        def _(i):
            zbuf[pl.ds(i * 16, 16)] = jnp.zeros((16,), jnp.float32)

        pltpu.sync_copy(zbuf, acc.at[pl.ds(sid * RP_SC, RP_SC)])
        plsc.subcore_barrier()

        @pl.loop(0, NSUP, step=2)
        def _(s0):
            for b in range(2):
                s = s0 + b
                pltpu.make_async_copy(
                    col_hbm.at[pl.ds(ibase + s * SUPR, SUPR)], cix.at[b], isem
                ).wait()
                for j in range(SUPR):
                    pltpu.sync_copy(ones, acc.at[cix.at[b].at[j]], add=True)

                @pl.when(s + 2 < NSUP)
                def _():
                    fire(b, s + 2)

        plsc.subcore_barrier()
        pltpu.sync_copy(
            acc.at[pl.ds(sid * RP_SC, RP_SC)],
            out_hbm.at[cid].at[pl.ds(sid * RP_SC, RP_SC)],
        )

    return k(col2d)


def _edge_layer(hs, row2d, col2d, D):
    @functools.partial(
        pl.kernel,
        out_type=jax.ShapeDtypeStruct((2, NP, D), jnp.float32),
        mesh=_mesh,
        compiler_params=_sc_params,
        scratch_types=[
            pltpu.VMEM((2, CHR, 128), jnp.int32),
            pltpu.VMEM((2, CHR, 128), jnp.int32),
            pltpu.VMEM((2, CHR * 128, D), jnp.float32),
            pltpu.VMEM((128, D), jnp.float32),
            pltpu.VMEM_SHARED((NP, D), jnp.float32),
            pltpu.SemaphoreType.DMA,
            pltpu.SemaphoreType.DMA,
        ],
    )
    def k(hs_hbm, row_hbm, col_hbm, out_hbm, rix, cix, msg, zblk, acc, gsem,
          zsem):
        cid = lax.axis_index("c")
        sid = lax.axis_index("s")
        tid = cid * 16 + sid
        rbase = sid * RP_SC
        ibase = tid * ROWS_PT

        def fire(slot, s):
            ro = ibase + s * CHR
            pltpu.sync_copy(row_hbm.at[pl.ds(ro, CHR)], rix.at[slot])
            pltpu.sync_copy(col_hbm.at[pl.ds(ro, CHR)], cix.at[slot])
            for j in range(CHR):
                pltpu.async_copy(
                    hs_hbm.at[rix.at[slot].at[j]],
                    msg.at[slot].at[pl.ds(j * 128, 128)],
                    gsem,
                )

        fire(0, 0)
        fire(1, 1)

        for i in range(128):
            for j in range(D // 16):
                zblk[i, pl.ds(j * 16, 16)] = jnp.zeros((16,), jnp.float32)

        for i in range(RP_SC // 128):
            pltpu.async_copy(zblk, acc.at[pl.ds(rbase + i * 128, 128)], zsem)
        for i in range(RP_SC // 128):
            pltpu.make_async_copy(
                zblk, acc.at[pl.ds(rbase + i * 128, 128)], zsem
            ).wait()

        plsc.subcore_barrier()

        @pl.loop(0, NCH, step=2)
        def _(s0):
            for b in range(2):
                s = s0 + b
                for j in range(CHR):
                    pltpu.make_async_copy(
                        hs_hbm.at[rix.at[b].at[j]],
                        msg.at[b].at[pl.ds(j * 128, 128)],
                        gsem,
                    ).wait()

                for j in range(CHR):
                    pltpu.sync_copy(
                        msg.at[b].at[pl.ds(j * 128, 128)],
                        acc.at[cix.at[b].at[j]],
                        add=True,
                    )

                @pl.when(s + 2 < NCH)
                def _():
                    fire(b, s + 2)

        plsc.subcore_barrier()
        pltpu.sync_copy(
            acc.at[pl.ds(rbase, RP_SC)],
            out_hbm.at[cid].at[pl.ds(rbase, RP_SC)],
        )

    return k(hs, row2d, col2d)


def _dis_of(degp):
    d = degp[0, :N, :] + degp[1, :N, :] + 1.0
    return lax.rsqrt(d)


def _mmul(a, b):
    return jnp.dot(a, b, preferred_element_type=jnp.float32)


def _bnorm(h, g, b):
    m = jnp.mean(h, axis=0, keepdims=True)
    hc = h - m
    v = jnp.mean(hc * hc, axis=0, keepdims=True)
    return hc * lax.rsqrt(v + 1e-5) * g + b


def _tc_mm0(x, w):
    def body(x_ref, w_ref, o_ref):
        o_ref[...] = _mmul(x_ref[...], w_ref[...])

    return pl.pallas_call(
        body,
        compiler_params=_tc_params,
        out_shape=jax.ShapeDtypeStruct((x.shape[0], w.shape[1]), jnp.float32),
    )(x, w)


def _tc_scale0(hw0, degp):
    def body(hw_ref, dg_ref, o_ref):
        dis = _dis_of(dg_ref[...])
        o_ref[...] = hw_ref[...] * dis

    return pl.pallas_call(
        body,
        compiler_params=_tc_params,
        out_shape=jax.ShapeDtypeStruct(hw0.shape, jnp.float32),
    )(hw0, degp)


def _tc_layer(part, hw, degp, b, g, beta, w_next):
    def body(p_ref, hw_ref, dg_ref, b_ref, g_ref, be_ref, w_ref, hs_o, hw_o):
        dis = _dis_of(dg_ref[...])
        scat = p_ref[0, :N, :] + p_ref[1, :N, :]
        pre = dis * scat + (dis * dis) * hw_ref[...] + b_ref[...]
        h = _bnorm(jnp.maximum(pre, 0.0), g_ref[...], be_ref[...])
        hw_n = _mmul(h, w_ref[...])
        hw_o[...] = hw_n
        hs_o[...] = dis * hw_n

    F = w_next.shape[1]
    return pl.pallas_call(
        body,
        compiler_params=_tc_params,
        out_shape=(
            jax.ShapeDtypeStruct((N, F), jnp.float32),
            jax.ShapeDtypeStruct((N, F), jnp.float32),
        ),
    )(part, hw, degp, b.reshape(1, -1), g.reshape(1, -1), beta.reshape(1, -1),
      w_next)


def _tc_tail(part, hw, degp, b, g, beta, ws):
    (lat_W, lat_b, dec_W0, dec_b0, dg0, db0, dec_W1, dec_b1, dg1, db1,
     dec_W2, dec_b2, dg2, db2, out_W, out_b) = ws

    def body(p_ref, hw_ref, dg_ref, b_ref, g_ref, be_ref,
             lw_ref, lb_ref, w0_ref, c0_ref, g0_ref, be0_ref,
             w1_ref, c1_ref, g1_ref, be1_ref,
             w2_ref, c2_ref, g2_ref, be2_ref, ow_ref, ob_ref, o_ref):
        dis = _dis_of(dg_ref[...])
        scat = p_ref[0, :N, :] + p_ref[1, :N, :]
        pre = dis * scat + (dis * dis) * hw_ref[...] + b_ref[...]
        h = _bnorm(jnp.maximum(pre, 0.0), g_ref[...], be_ref[...])
        z = _mmul(h, lw_ref[...]) + lb_ref[...]
        d = _bnorm(jnp.maximum(_mmul(z, w0_ref[...]) + c0_ref[...], 0.0),
                   g0_ref[...], be0_ref[...])
        d = _bnorm(jnp.maximum(_mmul(d, w1_ref[...]) + c1_ref[...], 0.0),
                   g1_ref[...], be1_ref[...])
        d = _bnorm(jnp.maximum(_mmul(d, w2_ref[...]) + c2_ref[...], 0.0),
                   g2_ref[...], be2_ref[...])
        o_ref[...] = _mmul(d, ow_ref[...]) + ob_ref[...]

    r = lambda a: a.reshape(1, -1)
    return pl.pallas_call(
        body,
        compiler_params=_tc_params,
        out_shape=jax.ShapeDtypeStruct((N, out_W.shape[1]), jnp.float32),
    )(part, hw, degp, r(b), r(g), r(beta),
      lat_W, r(lat_b), dec_W0, r(dec_b0), r(dg0), r(db0),
      dec_W1, r(dec_b1), r(dg1), r(db1),
      dec_W2, r(dec_b2), r(dg2), r(db2), out_W, r(out_b))


def kernel(x, edge_index, enc_W0, enc_b0, bn_g0, bn_b0, enc_W1, enc_b1,
           bn_g1, bn_b1, enc_W2, enc_b2, bn_g2, bn_b2, lat_W, lat_b,
           dec_W0, dec_b0, dbn_g0, dbn_b0, dec_W1, dec_b1, dbn_g1, dbn_b1,
           dec_W2, dec_b2, dbn_g2, dbn_b2, out_W, out_b):
    npad = EP - E
    pi = jnp.arange(npad, dtype=jnp.int32)
    row = jnp.concatenate([edge_index[0], pi % N])
    col = jnp.concatenate([edge_index[1], N + pi % (NP - N)])
    row2d = row.reshape(EP // 128, 128)
    col2d = col.reshape(EP // 128, 128)

    degp = _deg_kernel(col2d)
    hw0 = _tc_mm0(x, enc_W0)
    degp = degp.reshape(2, NP, 1)

    hs0 = _tc_scale0(hw0, degp)
    p0 = _edge_layer(hs0, row2d, col2d, 64)
    hs1, hw1 = _tc_layer(p0, hw0, degp, enc_b0, bn_g0, bn_b0, enc_W1)
    p1 = _edge_layer(hs1, row2d, col2d, 32)
    hs2, hw2 = _tc_layer(p1, hw1, degp, enc_b1, bn_g1, bn_b1, enc_W2)
    p2 = _edge_layer(hs2, row2d, col2d, 16)
    return _tc_tail(p2, hw2, degp, enc_b2, bn_g2, bn_b2,
                    (lat_W, lat_b, dec_W0, dec_b0, dbn_g0, dbn_b0,
                     dec_W1, dec_b1, dbn_g1, dbn_b1,
                     dec_W2, dec_b2, dbn_g2, dbn_b2, out_W, out_b))
